# baseline (speedup 1.0000x reference)
import functools

import jax
import jax.numpy as jnp
from jax import lax
from jax.experimental import pallas as pl
from jax.experimental.pallas import tpu as pltpu
from jax.experimental.pallas import tpu_sc as plsc

_L = 16
_NCLS = 1024
_BATCH = 16384
_NC = 2
_NSUB = 16
_NW = _NC * _NSUB
_CHUNK = _BATCH // _NW
_VECS = _CHUNK // _L

_mesh = plsc.VectorSubcoreMesh(
    core_axis_name="c", subcore_axis_name="s", num_cores=_NC)


@functools.partial(
    pl.kernel,
    out_type=jax.ShapeDtypeStruct((_NW, 2 * _NCLS), jnp.float32),
    mesh=_mesh,
    compiler_params=pltpu.CompilerParams(needs_layout_passes=False),
    scratch_types=[
        pltpu.VMEM((_CHUNK,), jnp.int32),
        pltpu.VMEM((_CHUNK,), jnp.int32),
        pltpu.VMEM((2 * _NCLS,), jnp.float32),
        pltpu.SemaphoreType.DMA,
    ],
)
def _precision_hist(yt_hbm, yp_hbm, part_hbm, predv, truev, histv, dsem):
    wid = lax.axis_index("s") * _NC + lax.axis_index("c")
    base = wid * _CHUNK

    in_cp1 = pltpu.async_copy(yp_hbm.at[pl.ds(base, _CHUNK)], predv, dsem)
    in_cp2 = pltpu.async_copy(yt_hbm.at[pl.ds(base, _CHUNK)], truev, dsem)

    zeros = jnp.zeros((_L,), jnp.float32)

    @plsc.parallel_loop(0, 2 * _NCLS // _L, unroll=8)
    def _zero_body(i):
        histv[pl.ds(i * _L, _L)] = zeros

    in_cp1.wait()
    in_cp2.wait()

    @plsc.parallel_loop(0, _VECS, unroll=4)
    def _hist_body(i):
        p = predv[pl.ds(i * _L, _L)]
        t = truev[pl.ds(i * _L, _L)]
        rc_all, last_all = plsc.scan_count(p)
        plsc.addupdate_scatter(
            histv, [p], rc_all.astype(jnp.float32), mask=last_all)
        rc_tp, last_tp = plsc.scan_count(p, mask=p == t)
        plsc.addupdate_scatter(
            histv, [p + _NCLS], rc_tp.astype(jnp.float32), mask=last_tp)

    pltpu.sync_copy(histv, part_hbm.at[wid])


def _combine_body(part_ref, out_ref):
    part = part_ref[...]
    cnt = jnp.sum(part[:, :_NCLS], axis=0, keepdims=True)
    tp = jnp.sum(part[:, _NCLS:], axis=0, keepdims=True)
    valid = cnt > 0.0
    prec = tp / jnp.where(valid, cnt, 1.0)
    s_tot = jnp.sum(jnp.where(valid, prec, 0.0))
    n_tot = jnp.sum(jnp.where(valid, 1.0, 0.0))
    out_ref[...] = jnp.full((1, 1), s_tot / n_tot, jnp.float32)


_precision_combine = pl.pallas_call(
    _combine_body,
    out_shape=jax.ShapeDtypeStruct((1, 1), jnp.float32),
)


def kernel(y_true, y_pred):
    part = _precision_hist(y_true, y_pred)
    out = _precision_combine(part)
    return out[0, 0]


# speedup vs baseline: 1.0653x; 1.0653x over previous
import functools

import jax
import jax.numpy as jnp
from jax import lax
from jax.experimental import pallas as pl
from jax.experimental.pallas import tpu as pltpu
from jax.experimental.pallas import tpu_sc as plsc

_L = 16
_NCLS = 1024
_BATCH = 16384
_NC = 1
_NSUB = 16
_NW = _NC * _NSUB
_CHUNK = _BATCH // _NW
_VECS = _CHUNK // _L

_mesh = plsc.VectorSubcoreMesh(
    core_axis_name="c", subcore_axis_name="s", num_cores=_NC)


@functools.partial(
    pl.kernel,
    out_type=jax.ShapeDtypeStruct((_NW, 2 * _NCLS), jnp.float32),
    mesh=_mesh,
    compiler_params=pltpu.CompilerParams(needs_layout_passes=False),
    scratch_types=[
        pltpu.VMEM((_CHUNK,), jnp.int32),
        pltpu.VMEM((_CHUNK,), jnp.int32),
        pltpu.VMEM((2 * _NCLS,), jnp.float32),
        pltpu.SemaphoreType.DMA,
    ],
)
def _precision_hist(yt_hbm, yp_hbm, part_hbm, predv, truev, histv, dsem):
    wid = lax.axis_index("s") * _NC + lax.axis_index("c")
    base = wid * _CHUNK

    in_cp1 = pltpu.async_copy(yp_hbm.at[pl.ds(base, _CHUNK)], predv, dsem)
    in_cp2 = pltpu.async_copy(yt_hbm.at[pl.ds(base, _CHUNK)], truev, dsem)

    zeros = jnp.zeros((_L,), jnp.float32)

    @plsc.parallel_loop(0, 2 * _NCLS // _L, unroll=8)
    def _zero_body(i):
        histv[pl.ds(i * _L, _L)] = zeros

    in_cp1.wait()
    in_cp2.wait()

    @plsc.parallel_loop(0, _VECS, unroll=4)
    def _hist_body(i):
        p = predv[pl.ds(i * _L, _L)]
        t = truev[pl.ds(i * _L, _L)]
        rc_all, last_all = plsc.scan_count(p)
        plsc.addupdate_scatter(
            histv, [p], rc_all.astype(jnp.float32), mask=last_all)
        rc_tp, last_tp = plsc.scan_count(p, mask=p == t)
        plsc.addupdate_scatter(
            histv, [p + _NCLS], rc_tp.astype(jnp.float32), mask=last_tp)

    pltpu.sync_copy(histv, part_hbm.at[wid])


def _combine_body(part_ref, out_ref):
    part = part_ref[...]
    cnt = jnp.sum(part[:, :_NCLS], axis=0, keepdims=True)
    tp = jnp.sum(part[:, _NCLS:], axis=0, keepdims=True)
    valid = cnt > 0.0
    prec = tp / jnp.where(valid, cnt, 1.0)
    s_tot = jnp.sum(jnp.where(valid, prec, 0.0))
    n_tot = jnp.sum(jnp.where(valid, 1.0, 0.0))
    out_ref[...] = jnp.full((1, 1), s_tot / n_tot, jnp.float32)


_precision_combine = pl.pallas_call(
    _combine_body,
    out_shape=jax.ShapeDtypeStruct((1, 1), jnp.float32),
)


def kernel(y_true, y_pred):
    part = _precision_hist(y_true, y_pred)
    out = _precision_combine(part)
    return out[0, 0]


# speedup vs baseline: 1.0730x; 1.0072x over previous
import functools

import jax
import jax.numpy as jnp
from jax import lax
from jax.experimental import pallas as pl
from jax.experimental.pallas import tpu as pltpu
from jax.experimental.pallas import tpu_sc as plsc

_L = 16
_NCLS = 1024
_BATCH = 16384
_NC = 1
_NSUB = 16
_NW = _NC * _NSUB
_CHUNK = _BATCH // _NW
_VECS = _CHUNK // _L

_mesh = plsc.VectorSubcoreMesh(
    core_axis_name="c", subcore_axis_name="s", num_cores=_NC)


@functools.partial(
    pl.kernel,
    out_type=jax.ShapeDtypeStruct((_NW, 2 * _NCLS), jnp.int32),
    mesh=_mesh,
    compiler_params=pltpu.CompilerParams(needs_layout_passes=False),
    scratch_types=[
        pltpu.VMEM((_CHUNK,), jnp.int32),
        pltpu.VMEM((_CHUNK,), jnp.int32),
        pltpu.VMEM((2 * _NCLS,), jnp.int32),
        pltpu.SemaphoreType.DMA,
    ],
)
def _precision_hist(yt_hbm, yp_hbm, part_hbm, predv, truev, histv, dsem):
    wid = lax.axis_index("s") * _NC + lax.axis_index("c")
    base = wid * _CHUNK

    in_cp1 = pltpu.async_copy(yp_hbm.at[pl.ds(base, _CHUNK)], predv, dsem)
    in_cp2 = pltpu.async_copy(yt_hbm.at[pl.ds(base, _CHUNK)], truev, dsem)

    zeros = jnp.zeros((_L,), jnp.int32)

    @plsc.parallel_loop(0, 2 * _NCLS // _L, unroll=8)
    def _zero_body(i):
        histv[pl.ds(i * _L, _L)] = zeros

    in_cp1.wait()
    in_cp2.wait()

    @plsc.parallel_loop(0, _VECS, unroll=4)
    def _hist_body(i):
        p = predv[pl.ds(i * _L, _L)]
        t = truev[pl.ds(i * _L, _L)]
        rc_all, last_all = plsc.scan_count(p)
        plsc.addupdate_scatter(histv, [p], rc_all, mask=last_all)
        rc_tp, last_tp = plsc.scan_count(p, mask=p == t)
        plsc.addupdate_scatter(histv, [p + _NCLS], rc_tp, mask=last_tp)

    pltpu.sync_copy(histv, part_hbm.at[wid])


def _combine_body(part_ref, out_ref):
    part = part_ref[...].astype(jnp.float32)
    cnt = jnp.sum(part[:, :_NCLS], axis=0, keepdims=True)
    tp = jnp.sum(part[:, _NCLS:], axis=0, keepdims=True)
    valid = cnt > 0.0
    prec = tp / jnp.where(valid, cnt, 1.0)
    s_tot = jnp.sum(jnp.where(valid, prec, 0.0))
    n_tot = jnp.sum(jnp.where(valid, 1.0, 0.0))
    out_ref[...] = jnp.full((1, 1), s_tot / n_tot, jnp.float32)


_precision_combine = pl.pallas_call(
    _combine_body,
    out_shape=jax.ShapeDtypeStruct((1, 1), jnp.float32),
)


def kernel(y_true, y_pred):
    part = _precision_hist(y_true, y_pred)
    out = _precision_combine(part)
    return out[0, 0]
